# baseline (speedup 1.0000x reference)
import functools

import jax
import jax.numpy as jnp
from jax import lax
from jax.experimental import pallas as pl
from jax.experimental.pallas import tpu as pltpu
from jax.experimental.pallas import tpu_sc as plsc

EMB = 64
BATCH = 16384
NUM_CORES = 2
NUM_SUBCORES = 16
LANES = 16
VECS = EMB // LANES
NUM_WORKERS = NUM_CORES * NUM_SUBCORES
ROWS_PER_W = BATCH // NUM_WORKERS
IDX_CHUNK = 128
NUM_CHUNKS = ROWS_PER_W // IDX_CHUNK
GROUP = 16
TROW = 2 * EMB


def _decode_body(actions_hbm, table2_hbm, state_t_hbm, w4_hbm, w5_hbm,
                 out_hbm, idx_v, idx2_v, za_v, st_v, w4_v, w5_v, s_v, q_v,
                 *sems):
    wid = lax.axis_index("s") * NUM_CORES + lax.axis_index("c")
    base = wid * ROWS_PER_W
    st_sem = sems[NUM_CHUNKS]

    pltpu.sync_copy(actions_hbm.at[pl.ds(wid * NUM_CHUNKS, NUM_CHUNKS)], idx_v)
    st_copy = pltpu.async_copy(
        state_t_hbm.at[:, pl.ds(base, ROWS_PER_W)], st_v, st_sem)

    def half_body(i, carry):
        k = i // (IDX_CHUNK // GROUP)
        g = i % (IDX_CHUNK // GROUP)
        a = idx_v[k, pl.ds(g * GROUP, GROUP)]
        idx2_v[k, pl.ds(g * GROUP, GROUP)] = lax.shift_right_logical(a, 1)
        return carry

    lax.fori_loop(0, NUM_CHUNKS * (IDX_CHUNK // GROUP), half_body, 0)

    copies = []
    for k in range(NUM_CHUNKS):
        copies.append(
            pltpu.async_copy(table2_hbm.at[idx2_v.at[k]],
                             za_v.at[pl.ds(k * IDX_CHUNK, IDX_CHUNK)], sems[k]))
    pltpu.sync_copy(w4_hbm, w4_v)
    pltpu.sync_copy(w5_hbm, w5_v)

    w4vecs = [w4_v[0, pl.ds(t * LANES, LANES)] for t in range(VECS)]
    w5vecs = [w5_v[0, pl.ds(t * LANES, LANES)] for t in range(VECS)]
    w4s = [w4vecs[j // LANES][j % LANES] for j in range(EMB)]
    zero = jnp.zeros((LANES,), jnp.float32)
    lane_iota = lax.iota(jnp.int32, LANES)

    st_copy.wait()

    def s_body(g, carry):
        col = g * LANES
        acc = st_v[0, pl.ds(col, LANES)] * w4s[0]
        for j in range(1, EMB):
            acc = acc + st_v[j, pl.ds(col, LANES)] * w4s[j]
        s_v[pl.ds(col, LANES)] = acc
        return carry

    lax.fori_loop(0, ROWS_PER_W // LANES, s_body, 0)

    for k in range(NUM_CHUNKS):
        copies[k].wait()

        def q_body(g, carry, k=k):
            grow = k * IDX_CHUNK + g * GROUP
            s_vec = s_v[pl.ds(grow, GROUP)]
            par_vec = idx_v[k, pl.ds(g * GROUP, GROUP)] & 1
            q_vec = zero
            for r in range(GROUP):
                row = grow + r
                s_r = s_vec[r]
                even = par_vec[r] == 0
                qacc = zero
                for t in range(VECS):
                    lo = za_v[row, pl.ds(t * LANES, LANES)]
                    hi = za_v[row, pl.ds(EMB + t * LANES, LANES)]
                    v = jnp.where(even, lo, hi)
                    qacc = qacc + jnp.maximum(v * s_r, 0.0) * w5vecs[t]
                q_r = jnp.sum(qacc)
                q_vec = jnp.where(lane_iota == r, q_r, q_vec)
            q_v[pl.ds(grow, GROUP)] = q_vec
            return carry

        lax.fori_loop(0, IDX_CHUNK // GROUP, q_body, 0)

    pltpu.sync_copy(q_v, out_hbm.at[pl.ds(base, ROWS_PER_W)])


@jax.jit
def _decode(actions2d, table2, state_t, w4, w5):
    mesh = plsc.VectorSubcoreMesh(core_axis_name="c", subcore_axis_name="s")
    return pl.kernel(
        _decode_body,
        mesh=mesh,
        compiler_params=pltpu.CompilerParams(needs_layout_passes=False),
        out_type=jax.ShapeDtypeStruct((BATCH,), jnp.float32),
        scratch_types=[
            pltpu.VMEM((NUM_CHUNKS, IDX_CHUNK), jnp.int32),
            pltpu.VMEM((NUM_CHUNKS, IDX_CHUNK), jnp.int32),
            pltpu.VMEM((ROWS_PER_W, TROW), jnp.float32),
            pltpu.VMEM((EMB, ROWS_PER_W), jnp.float32),
            pltpu.VMEM((1, EMB), jnp.float32),
            pltpu.VMEM((1, EMB), jnp.float32),
            pltpu.VMEM((ROWS_PER_W,), jnp.float32),
            pltpu.VMEM((ROWS_PER_W,), jnp.float32),
        ] + [pltpu.SemaphoreType.DMA] * (NUM_CHUNKS + 1),
    )(actions2d, table2, state_t, w4, w5)


def kernel(actions, node_embedding, state_embedding, W_4, W_5):
    actions2d = actions.astype(jnp.int32).reshape(BATCH // IDX_CHUNK, IDX_CHUNK)
    table2 = node_embedding.reshape(1000000 // 2, TROW)
    out = _decode(actions2d, table2, state_embedding.T,
                  W_4.reshape(1, EMB), W_5.reshape(1, EMB))
    return out.reshape(BATCH, 1)


# speedup vs baseline: 1.6245x; 1.6245x over previous
import functools

import jax
import jax.numpy as jnp
from jax import lax
from jax.experimental import pallas as pl
from jax.experimental.pallas import tpu as pltpu
from jax.experimental.pallas import tpu_sc as plsc

EMB = 64
BATCH = 16384
NUM_CORES = 2
NUM_SUBCORES = 16
LANES = 16
VECS = EMB // LANES
NUM_WORKERS = NUM_CORES * NUM_SUBCORES
ROWS_PER_W = BATCH // NUM_WORKERS
WAVE = 32
NUM_WAVES = ROWS_PER_W // WAVE
GROUPS_PER_WAVE = WAVE // LANES
SLAB = 8
HALF_ROWS = WAVE * SLAB


def _decode_body(actions_hbm, table_hbm, state_t_hbm, w4_hbm, w5_hbm,
                 out_hbm, idx_v, za_v, st_v, w4_v, w5_v, s_v, q_v,
                 sem_a, sem_b, st_sem):
    wid = lax.axis_index("s") * NUM_CORES + lax.axis_index("c")
    base = wid * ROWS_PER_W

    pltpu.sync_copy(actions_hbm.at[pl.ds(base, ROWS_PER_W)], idx_v)
    st_copy = pltpu.async_copy(
        state_t_hbm.at[:, pl.ds(base, ROWS_PER_W)], st_v, st_sem)
    pltpu.sync_copy(w4_hbm, w4_v)
    pltpu.sync_copy(w5_hbm, w5_v)

    def issue_wave(wi, half_off, sem):
        for gi in range(GROUPS_PER_WAVE):
            a_vec = idx_v[pl.ds(wi * WAVE + gi * LANES, LANES)]
            for r in range(LANES):
                src = pl.multiple_of(
                    lax.shift_right_logical(a_vec[r], 3) * SLAB, SLAB)
                dst = half_off + gi * LANES * SLAB + r * SLAB
                pltpu.async_copy(
                    table_hbm.at[pl.ds(src, SLAB), :],
                    za_v.at[pl.ds(dst, SLAB), :], sem)

    issue_wave(0, 0, sem_a)
    issue_wave(1, HALF_ROWS, sem_b)

    w4vecs = [w4_v[0, pl.ds(t * LANES, LANES)] for t in range(VECS)]
    w5vecs = [w5_v[0, pl.ds(t * LANES, LANES)] for t in range(VECS)]
    w4s = [w4vecs[j // LANES][j % LANES] for j in range(EMB)]
    zero = jnp.zeros((LANES,), jnp.float32)
    lane_iota = lax.iota(jnp.int32, LANES)

    st_copy.wait()

    def s_body(g, carry):
        col = g * LANES
        acc = st_v[0, pl.ds(col, LANES)] * w4s[0]
        for j in range(1, EMB):
            acc = acc + st_v[j, pl.ds(col, LANES)] * w4s[j]
        s_v[pl.ds(col, LANES)] = acc
        return carry

    lax.fori_loop(0, ROWS_PER_W // LANES, s_body, 0)

    def wave_body(w, carry):
        def run_half(half_off, sem):
            pltpu.make_async_copy(
                table_hbm.at[pl.ds(0, HALF_ROWS), :],
                za_v.at[pl.ds(half_off, HALF_ROWS), :], sem).wait()

            for gi in range(GROUPS_PER_WAVE):
                n0 = w * WAVE + gi * LANES
                a_vec = idx_v[pl.ds(n0, LANES)]
                s_vec = s_v[pl.ds(n0, LANES)]
                q_vec = zero
                for r in range(LANES):
                    row = half_off + gi * LANES * SLAB + r * SLAB + (a_vec[r] & 7)
                    s_r = s_vec[r]
                    qacc = jnp.maximum(
                        za_v[row, pl.ds(0, LANES)] * s_r, 0.0) * w5vecs[0]
                    for t in range(1, VECS):
                        qacc = qacc + jnp.maximum(
                            za_v[row, pl.ds(t * LANES, LANES)] * s_r,
                            0.0) * w5vecs[t]
                    q_r = jnp.sum(qacc)
                    q_vec = jnp.where(lane_iota == r, q_r, q_vec)
                q_v[pl.ds(n0, LANES)] = q_vec

            @pl.when(w + 2 < NUM_WAVES)
            def _():
                issue_wave(w + 2, half_off, sem)

        @pl.when(w % 2 == 0)
        def _():
            run_half(0, sem_a)

        @pl.when(w % 2 == 1)
        def _():
            run_half(HALF_ROWS, sem_b)

        return carry

    lax.fori_loop(0, NUM_WAVES, wave_body, 0)

    pltpu.sync_copy(q_v, out_hbm.at[pl.ds(base, ROWS_PER_W)])


@jax.jit
def _decode(actions1d, table, state_t, w4, w5):
    mesh = plsc.VectorSubcoreMesh(core_axis_name="c", subcore_axis_name="s")
    return pl.kernel(
        _decode_body,
        mesh=mesh,
        compiler_params=pltpu.CompilerParams(needs_layout_passes=False),
        out_type=jax.ShapeDtypeStruct((BATCH,), jnp.float32),
        scratch_types=[
            pltpu.VMEM((ROWS_PER_W,), jnp.int32),
            pltpu.VMEM((2 * HALF_ROWS, EMB), jnp.float32),
            pltpu.VMEM((EMB, ROWS_PER_W), jnp.float32),
            pltpu.VMEM((1, EMB), jnp.float32),
            pltpu.VMEM((1, EMB), jnp.float32),
            pltpu.VMEM((ROWS_PER_W,), jnp.float32),
            pltpu.VMEM((ROWS_PER_W,), jnp.float32),
            pltpu.SemaphoreType.DMA,
            pltpu.SemaphoreType.DMA,
            pltpu.SemaphoreType.DMA,
        ],
    )(actions1d, table, state_t, w4, w5)


def kernel(actions, node_embedding, state_embedding, W_4, W_5):
    out = _decode(actions.astype(jnp.int32), node_embedding,
                  state_embedding.T, W_4.reshape(1, EMB), W_5.reshape(1, EMB))
    return out.reshape(BATCH, 1)
